# baseline (speedup 1.0000x reference)
import jax
import jax.numpy as jnp
from jax import lax
from jax.experimental import pallas as pl
from jax.experimental.pallas import tpu as pltpu
from jax.experimental.pallas import tpu_sc as plsc

EMB_DIM = 64
NUM_CORES = 2
NUM_SUBCORES = 16
NUM_WORKERS = NUM_CORES * NUM_SUBCORES
CHUNK = 128
LANES = 16


def _transpose_block(rows_v, tb, iotas):
    def one_row(r, carry):
        col = jnp.full((LANES,), r, jnp.int32)
        for dh in range(EMB_DIM // LANES):
            v = rows_v[r, pl.ds(dh * LANES, LANES)]
            plsc.store_scatter(tb, [iotas[dh], col], v)
        return carry

    lax.fori_loop(0, CHUNK, one_row, 0, unroll=16)


def _sc_body(idx_hbm, w2_hbm, x_hbm, idx_v, rows_v, tb_v, g_sem, w_sem):
    n_tasks = idx_hbm.shape[1]
    wid = lax.axis_index("s") * NUM_CORES + lax.axis_index("c")
    t0 = wid * n_tasks
    pltpu.sync_copy(idx_hbm.at[wid], idx_v)

    iotas = [
        lax.iota(jnp.int32, LANES) + dh * LANES
        for dh in range(EMB_DIM // LANES)
    ]

    def start_gather(tl, b):
        pltpu.async_copy(w2_hbm.at[idx_v.at[tl]], rows_v.at[b], g_sem.at[b])

    def wait_gather(tl, b):
        pltpu.make_async_copy(
            w2_hbm.at[idx_v.at[tl]], rows_v.at[b], g_sem.at[b]
        ).wait()

    def tile_dst(t, dh):
        s = t // 32
        bh = t % 32
        return x_hbm.at[s, dh, bh]

    def start_writes(t, b):
        for dh in range(EMB_DIM // 8):
            pltpu.async_copy(
                tb_v.at[b, pl.ds(dh * 8, 8)], tile_dst(t, dh), w_sem.at[b]
            )

    def wait_writes(t, b):
        for dh in range(EMB_DIM // 8):
            pltpu.make_async_copy(
                tb_v.at[b, pl.ds(dh * 8, 8)], tile_dst(t, dh), w_sem.at[b]
            ).wait()

    start_gather(0, 0)
    start_gather(1, 1)

    def group(g, carry):
        for b in range(2):
            tl = g * 2 + b
            t = t0 + tl
            wait_gather(tl, b)
            @pl.when(g > 0)
            def _():
                wait_writes(t - 2, b)

            _transpose_block(rows_v.at[b], tb_v.at[b], iotas)
            start_writes(t, b)

            @pl.when(tl + 2 < n_tasks)
            def _():
                start_gather(tl + 2, b)

        return carry

    lax.fori_loop(0, n_tasks // 2, group, 0)
    wait_writes(t0 + n_tasks - 2, 0)
    wait_writes(t0 + n_tasks - 1, 1)


def kernel(words, weight):
    b, s = words.shape
    n_tiles_b = b // CHUNK
    n_tasks = s * n_tiles_b // NUM_WORKERS

    w2 = jnp.pad(weight, ((0, 0), (0, 64))).reshape(2 * weight.shape[0], 64)
    idx2 = (words.T * 2).reshape(NUM_WORKERS, n_tasks, CHUNK)

    mesh = plsc.VectorSubcoreMesh(core_axis_name="c", subcore_axis_name="s")
    x = pl.kernel(
        _sc_body,
        out_type=jax.ShapeDtypeStruct(
            (s, EMB_DIM // 8, n_tiles_b, 8, CHUNK), jnp.float32
        ),
        mesh=mesh,
        compiler_params=pltpu.CompilerParams(
            use_tc_tiling_on_sc=False, needs_layout_passes=False
        ),
        scratch_types=[
            pltpu.VMEM((n_tasks, CHUNK), jnp.int32),
            pltpu.VMEM((2, CHUNK, EMB_DIM), jnp.float32),
            pltpu.VMEM((2, EMB_DIM, CHUNK), jnp.float32),
            pltpu.SemaphoreType.DMA((2,)),
            pltpu.SemaphoreType.DMA((2,)),
        ],
    )(idx2, w2)
    return x.transpose(2, 4, 0, 1, 3).reshape(b, s, EMB_DIM)


# speedup vs baseline: 1.3518x; 1.3518x over previous
import jax
import jax.numpy as jnp
from jax import lax
from jax.experimental import pallas as pl
from jax.experimental.pallas import tpu as pltpu
from jax.experimental.pallas import tpu_sc as plsc

EMB_DIM = 64
NUM_CORES = 2
NUM_SUBCORES = 16
NUM_WORKERS = NUM_CORES * NUM_SUBCORES
CHUNK = 128
LANES = 16
TB_PITCH = 129


def _transpose_block(rows_v, tb, diotas):
    def one_row(r, carry):
        col = jnp.full((LANES,), r, jnp.int32)
        for dh in range(EMB_DIM // LANES):
            v = rows_v[r, pl.ds(dh * LANES, LANES)]
            plsc.store_scatter(tb, [diotas[dh], col], v)
        return carry

    lax.fori_loop(0, CHUNK, one_row, 0, unroll=8)


def _sc_body(idx_hbm, w2_hbm, x_hbm, idx_v, rows_v, tb_v, g_sem, w_sem):
    n_tasks = idx_hbm.shape[1]
    wid = lax.axis_index("s") * NUM_CORES + lax.axis_index("c")
    t0 = wid * n_tasks
    pltpu.sync_copy(idx_hbm.at[wid], idx_v)

    iota = lax.iota(jnp.int32, LANES)
    diotas = [iota + dh * LANES for dh in range(EMB_DIM // LANES)]

    def start_gather(tl, b):
        pltpu.async_copy(w2_hbm.at[idx_v.at[tl]], rows_v.at[b], g_sem.at[b])

    def wait_gather(tl, b):
        pltpu.make_async_copy(
            w2_hbm.at[idx_v.at[tl]], rows_v.at[b], g_sem.at[b]
        ).wait()

    def tile_dst(t, dh):
        s = t // 32
        bh = t % 32
        return x_hbm.at[s, dh, bh]

    def tile_src(b, dh):
        return tb_v.at[b, pl.ds(dh * 8, 8), pl.ds(0, CHUNK)]

    def start_writes(t, b):
        for dh in range(EMB_DIM // 8):
            pltpu.async_copy(tile_src(b, dh), tile_dst(t, dh), w_sem.at[b])

    def wait_writes(t, b):
        for dh in range(EMB_DIM // 8):
            pltpu.make_async_copy(
                tile_src(b, dh), tile_dst(t, dh), w_sem.at[b]
            ).wait()

    start_gather(0, 0)
    start_gather(1, 1)

    def group(g, carry):
        for b in range(2):
            tl = g * 2 + b
            t = t0 + tl
            wait_gather(tl, b)
            @pl.when(g > 0)
            def _():
                wait_writes(t - 2, b)

            _transpose_block(rows_v.at[b], tb_v.at[b], diotas)
            start_writes(t, b)

            @pl.when(tl + 2 < n_tasks)
            def _():
                start_gather(tl + 2, b)

        return carry

    lax.fori_loop(0, n_tasks // 2, group, 0)
    wait_writes(t0 + n_tasks - 2, 0)
    wait_writes(t0 + n_tasks - 1, 1)


def kernel(words, weight):
    b, s = words.shape
    n_tiles_b = b // CHUNK
    n_tasks = s * n_tiles_b // NUM_WORKERS

    w2 = jnp.pad(weight, ((0, 0), (0, 64))).reshape(2 * weight.shape[0], 64)
    idx2 = (words.T * 2).reshape(NUM_WORKERS, n_tasks, CHUNK)

    mesh = plsc.VectorSubcoreMesh(core_axis_name="c", subcore_axis_name="s")
    x = pl.kernel(
        _sc_body,
        out_type=jax.ShapeDtypeStruct(
            (s, EMB_DIM // 8, n_tiles_b, 8, CHUNK), jnp.float32
        ),
        mesh=mesh,
        compiler_params=pltpu.CompilerParams(
            use_tc_tiling_on_sc=False, needs_layout_passes=False
        ),
        scratch_types=[
            pltpu.VMEM((n_tasks, CHUNK), jnp.int32),
            pltpu.VMEM((2, CHUNK, EMB_DIM), jnp.float32),
            pltpu.VMEM((2, EMB_DIM, TB_PITCH), jnp.float32),
            pltpu.SemaphoreType.DMA((2,)),
            pltpu.SemaphoreType.DMA((2,)),
        ],
    )(idx2, w2)
    return x.transpose(2, 4, 0, 1, 3).reshape(b, s, EMB_DIM)
